# baseline (speedup 1.0000x reference)
import functools
import jax
import jax.numpy as jnp
from jax import lax
from jax.experimental import pallas as pl
from jax.experimental.pallas import tpu as pltpu
from jax.experimental.pallas import tpu_sc as plsc

_B, _S, _D = 4, 8192, 128
_E, _K, _H, _O = 64, 8, 512, 128
_NEG = float("-inf")


def _gelu(h):
    return 0.5 * h * (1.0 + jax.lax.erf(h * 0.7071067811865476))


def _gate_body(x_ref, wg_ref, bg_ref, idx_ref, tv_ref, inp_ref):
    x = x_ref[0]
    wg = wg_ref[...]
    logits = jax.lax.dot_general(
        wg, x, (((1,), (1,)), ((), ())), preferred_element_type=jnp.float32
    ) + bg_ref[...]
    m = jnp.max(logits, axis=1, keepdims=True)
    z = jnp.sum(jnp.exp(logits - m), axis=1, keepdims=True)
    zinv = 1.0 / z
    iota = jax.lax.broadcasted_iota(jnp.int32, (_E, _S), 1)
    k16 = jax.lax.broadcasted_iota(jnp.int32, (_E, 16), 1)
    lw = logits
    p = jnp.zeros((_E, _S), jnp.float32)
    idxa = jnp.zeros((_E, 16), jnp.int32)
    tva = jnp.zeros((_E, 16), jnp.float32)
    for k in range(_K):
        cm = jnp.max(lw, axis=1, keepdims=True)
        cidx = jnp.min(jnp.where(lw == cm, iota, _S), axis=1, keepdims=True)
        hit = iota == cidx
        tv = jnp.exp(cm - m) * zinv
        p = jnp.where(hit, tv, p)
        lw = jnp.where(hit, _NEG, lw)
        idxa = jnp.where(k16 == k, cidx, idxa)
        tva = jnp.where(k16 == k, tv, tva)
    idx_ref[0] = idxa
    tv_ref[0] = tva
    inp_ref[0] = jax.lax.dot_general(
        p, x, (((1,), (0,)), ((), ())), preferred_element_type=jnp.float32
    )


_EB = 8


def _ffn_body(inp_ref, tv_ref, w1_ref, w2_ref, srows_ref):
    for e in range(_EB):
        v = inp_ref[:, e, :]
        w1 = w1_ref[e]
        h = jnp.dot(v, w1[:_D], preferred_element_type=jnp.float32) + w1[_D:_D + 1]
        h = _gelu(h)
        w2 = w2_ref[e]
        o = jnp.dot(h, w2[:_H], preferred_element_type=jnp.float32) + w2[_H:_H + 1]
        s = tv_ref[:, e, :]
        srows_ref[:, e] = s[:, :, None] * o[:, None, :]


def _scatter_body(idx_ref, srows_ref, y_ref, zbuf, idxbuf, crows, ybuf):
    cid = lax.axis_index("c")
    sid = lax.axis_index("s")

    def zr(t, c):
        r = lax.shift_right_logical(t, 3)
        q = lax.bitwise_and(t, 7)
        zbuf[r, pl.ds(q * 16, 16)] = jnp.zeros((16,), jnp.float32)
        return c
    lax.fori_loop(0, 128 * (_O // 16), zr, 0)

    for bb_local in range(2):
        bb = cid * 2 + bb_local
        colbase = bb * _E + sid * 4
        pltpu.sync_copy(idx_ref.at[pl.ds(colbase * 16, 64)], idxbuf)
        pltpu.sync_copy(srows_ref.at[pl.ds(colbase * 16, 64)], crows)
        for t in range(4):
            pltpu.sync_copy(zbuf, ybuf.at[pl.ds(sid * 512 + t * 128, 128)])
        plsc.subcore_barrier()
        pltpu.sync_copy(crows, ybuf.at[idxbuf], add=True)
        plsc.subcore_barrier()
        pltpu.sync_copy(ybuf.at[pl.ds(sid * 512, 512)],
                        y_ref.at[pl.ds(bb * _S + sid * 512, 512)])
        plsc.subcore_barrier()


def _sc_scatter(idx_flat, srows_flat):
    mesh = plsc.VectorSubcoreMesh(core_axis_name="c", subcore_axis_name="s")
    fn = functools.partial(
        pl.kernel,
        mesh=mesh,
        out_type=jax.ShapeDtypeStruct((_B * _S, _O), jnp.float32),
        scratch_types=[
            pltpu.VMEM((128, _O), jnp.float32),
            pltpu.VMEM((64,), jnp.int32),
            pltpu.VMEM((64, _O), jnp.float32),
            pltpu.VMEM_SHARED((_S, _O), jnp.float32),
        ],
    )(_scatter_body)
    return fn(idx_flat, srows_flat)


def kernel(x, W_gate, b_gate, weight1, weight2):
    bg2 = b_gate.reshape(_E, 1)
    idx, tv, inp = pl.pallas_call(
        _gate_body,
        grid=(_B,),
        in_specs=[
            pl.BlockSpec((1, _S, _D), lambda b: (b, 0, 0)),
            pl.BlockSpec((_E, _D), lambda b: (0, 0)),
            pl.BlockSpec((_E, 1), lambda b: (0, 0)),
        ],
        out_specs=[
            pl.BlockSpec((1, _E, 16), lambda b: (b, 0, 0)),
            pl.BlockSpec((1, _E, 16), lambda b: (b, 0, 0)),
            pl.BlockSpec((1, _E, _D), lambda b: (b, 0, 0)),
        ],
        out_shape=[
            jax.ShapeDtypeStruct((_B, _E, 16), jnp.int32),
            jax.ShapeDtypeStruct((_B, _E, 16), jnp.float32),
            jax.ShapeDtypeStruct((_B, _E, _D), jnp.float32),
        ],
    )(x, W_gate, bg2)

    srows = pl.pallas_call(
        _ffn_body,
        grid=(_E // _EB,),
        in_specs=[
            pl.BlockSpec((_B, _EB, _D), lambda e: (0, e, 0)),
            pl.BlockSpec((_B, _EB, 16), lambda e: (0, e, 0)),
            pl.BlockSpec((_EB, _D + 1, _H), lambda e: (e, 0, 0)),
            pl.BlockSpec((_EB, _H + 1, _O), lambda e: (e, 0, 0)),
        ],
        out_specs=pl.BlockSpec((_B, _EB, 16, _O), lambda e: (0, e, 0, 0)),
        out_shape=jax.ShapeDtypeStruct((_B, _E, 16, _O), jnp.float32),
    )(inp, tv, weight1, weight2)

    y = _sc_scatter(
        idx.reshape(_B * _E * 16),
        srows.reshape(_B * _E * 16, _O),
    )
    return y.reshape(_B, _S, _O)


# speedup vs baseline: 1.0090x; 1.0090x over previous
import functools
import jax
import jax.numpy as jnp
from jax import lax
from jax.experimental import pallas as pl
from jax.experimental.pallas import tpu as pltpu
from jax.experimental.pallas import tpu_sc as plsc

_B, _S, _D = 4, 8192, 128
_E, _K, _H, _O = 64, 8, 512, 128
_NEG = float("-inf")


def _gelu(h):
    return 0.5 * h * (1.0 + jax.lax.erf(h * 0.7071067811865476))


def _gate_body(x_ref, wg_ref, bg_ref, idx_ref, tv_ref, inp_ref):
    x = x_ref[0]
    wg = wg_ref[...]
    logits = jax.lax.dot_general(
        wg, x, (((1,), (1,)), ((), ())), preferred_element_type=jnp.float32
    ) + bg_ref[...]
    m = jnp.max(logits, axis=1, keepdims=True)
    z = jnp.sum(jnp.exp(logits - m), axis=1, keepdims=True)
    zinv = 1.0 / z
    iota = jax.lax.broadcasted_iota(jnp.int32, (_E, _S), 1)
    k16 = jax.lax.broadcasted_iota(jnp.int32, (_E, 16), 1)
    lw = logits
    p = jnp.zeros((_E, _S), jnp.float32)
    idxa = jnp.zeros((_E, 16), jnp.int32)
    tva = jnp.zeros((_E, 16), jnp.float32)
    for k in range(_K):
        cm = jnp.max(lw, axis=1, keepdims=True)
        cidx = jnp.min(jnp.where(lw == cm, iota, _S), axis=1, keepdims=True)
        hit = iota == cidx
        tv = jnp.exp(cm - m) * zinv
        p = jnp.where(hit, tv, p)
        lw = jnp.where(hit, _NEG, lw)
        idxa = jnp.where(k16 == k, cidx, idxa)
        tva = jnp.where(k16 == k, tv, tva)
    idx_ref[0] = idxa
    tv_ref[0] = tva
    inp_ref[0] = jax.lax.dot_general(
        p, x, (((1,), (0,)), ((), ())), preferred_element_type=jnp.float32
    )


_EB = 16


def _ffn_body(inp_ref, tv_ref, w1_ref, w2_ref, srows_ref):
    for e in range(_EB):
        v = inp_ref[:, e, :]
        w1 = w1_ref[e]
        h = jnp.dot(v, w1[:_D], preferred_element_type=jnp.float32) + w1[_D:_D + 1]
        h = _gelu(h)
        w2 = w2_ref[e]
        o = jnp.dot(h, w2[:_H], preferred_element_type=jnp.float32) + w2[_H:_H + 1]
        s = tv_ref[:, e, :]
        srows_ref[:, e] = s[:, :, None] * o[:, None, :]


def _scatter_body(idx_ref, srows_ref, y_ref, zbuf, idxbuf, crows, ybuf):
    cid = lax.axis_index("c")
    sid = lax.axis_index("s")

    def zr(t, c):
        r = lax.shift_right_logical(t, 3)
        q = lax.bitwise_and(t, 7)
        zbuf[r, pl.ds(q * 16, 16)] = jnp.zeros((16,), jnp.float32)
        return c
    lax.fori_loop(0, 128 * (_O // 16), zr, 0)

    for bb_local in range(2):
        bb = cid * 2 + bb_local
        colbase = bb * _E + sid * 4
        pltpu.sync_copy(idx_ref.at[pl.ds(colbase * 16, 64)], idxbuf)
        pltpu.sync_copy(srows_ref.at[pl.ds(colbase * 16, 64)], crows)
        for t in range(4):
            pltpu.sync_copy(zbuf, ybuf.at[pl.ds(sid * 512 + t * 128, 128)])
        plsc.subcore_barrier()
        pltpu.sync_copy(crows, ybuf.at[idxbuf], add=True)
        plsc.subcore_barrier()
        pltpu.sync_copy(ybuf.at[pl.ds(sid * 512, 512)],
                        y_ref.at[pl.ds(bb * _S + sid * 512, 512)])
        plsc.subcore_barrier()


def _sc_scatter(idx_flat, srows_flat):
    mesh = plsc.VectorSubcoreMesh(core_axis_name="c", subcore_axis_name="s")
    fn = functools.partial(
        pl.kernel,
        mesh=mesh,
        out_type=jax.ShapeDtypeStruct((_B * _S, _O), jnp.float32),
        scratch_types=[
            pltpu.VMEM((128, _O), jnp.float32),
            pltpu.VMEM((64,), jnp.int32),
            pltpu.VMEM((64, _O), jnp.float32),
            pltpu.VMEM_SHARED((_S, _O), jnp.float32),
        ],
    )(_scatter_body)
    return fn(idx_flat, srows_flat)


def kernel(x, W_gate, b_gate, weight1, weight2):
    bg2 = b_gate.reshape(_E, 1)
    idx, tv, inp = pl.pallas_call(
        _gate_body,
        grid=(_B,),
        in_specs=[
            pl.BlockSpec((1, _S, _D), lambda b: (b, 0, 0)),
            pl.BlockSpec((_E, _D), lambda b: (0, 0)),
            pl.BlockSpec((_E, 1), lambda b: (0, 0)),
        ],
        out_specs=[
            pl.BlockSpec((1, _E, 16), lambda b: (b, 0, 0)),
            pl.BlockSpec((1, _E, 16), lambda b: (b, 0, 0)),
            pl.BlockSpec((1, _E, _D), lambda b: (b, 0, 0)),
        ],
        out_shape=[
            jax.ShapeDtypeStruct((_B, _E, 16), jnp.int32),
            jax.ShapeDtypeStruct((_B, _E, 16), jnp.float32),
            jax.ShapeDtypeStruct((_B, _E, _D), jnp.float32),
        ],
    )(x, W_gate, bg2)

    srows = pl.pallas_call(
        _ffn_body,
        grid=(_E // _EB,),
        in_specs=[
            pl.BlockSpec((_B, _EB, _D), lambda e: (0, e, 0)),
            pl.BlockSpec((_B, _EB, 16), lambda e: (0, e, 0)),
            pl.BlockSpec((_EB, _D + 1, _H), lambda e: (e, 0, 0)),
            pl.BlockSpec((_EB, _H + 1, _O), lambda e: (e, 0, 0)),
        ],
        out_specs=pl.BlockSpec((_B, _EB, 16, _O), lambda e: (0, e, 0, 0)),
        out_shape=jax.ShapeDtypeStruct((_B, _E, 16, _O), jnp.float32),
    )(inp, tv, weight1, weight2)

    y = _sc_scatter(
        idx.reshape(_B * _E * 16),
        srows.reshape(_B * _E * 16, _O),
    )
    return y.reshape(_B, _S, _O)
